# baseline (speedup 1.0000x reference)
import jax
import jax.numpy as jnp
from jax import lax
from jax.experimental import pallas as pl
from jax.experimental.pallas import tpu as pltpu

TOKENS = 32768
HIDDEN = 2048
FEAT_LEN = 1024
NUM_FEATS = 4
EMB_CHUNK = 4096
NCHUNK = TOKENS // EMB_CHUNK
SPAN = FEAT_LEN + 8
CC = 128


def _body(locs_ref, emb_ref, f0, f1, f2, f3, out_ref, u_ref, s_ref, semA, semB):
    feats = [f0, f1, f2, f3]
    copies = []
    for c in range(NCHUNK):
        cp = pltpu.make_async_copy(
            emb_ref.at[pl.ds(c * EMB_CHUNK, EMB_CHUNK)],
            out_ref.at[pl.ds(c * EMB_CHUNK, EMB_CHUNK)],
            semA.at[c],
        )
        cp.start()
        copies.append(cp)
    for cp in copies:
        cp.wait()

    for i in range(NUM_FEATS):
        loc = locs_ref[i]
        base = pl.multiple_of((loc // 8) * 8, 8)
        r = loc - base
        c0 = pltpu.make_async_copy(out_ref.at[pl.ds(base, 8)], u_ref.at[pl.ds(0, 8)], semB)
        c1 = pltpu.make_async_copy(feats[i], u_ref.at[pl.ds(8, FEAT_LEN)], semB)
        c2 = pltpu.make_async_copy(
            out_ref.at[pl.ds(base + FEAT_LEN, 8)], u_ref.at[pl.ds(SPAN, 8)], semB
        )
        c0.start(); c1.start(); c2.start()
        c0.wait(); c1.wait(); c2.wait()

        for k in range(FEAT_LEN // CC):
            c = k * CC
            u = u_ref[pl.ds(c, CC + 16), :]
            v = pltpu.roll(u, (CC + 8) + r, 0)
            if k == 0:
                rows = lax.broadcasted_iota(jnp.int32, (CC + 16, 1), 0)
                v = jnp.where(rows < r, u, v)
            s_ref[pl.ds(c, CC), :] = v[:CC, :]
        u16 = u_ref[pl.ds(FEAT_LEN, 16), :]
        v = pltpu.roll(u16, 8 + r, 0)
        rows8 = lax.broadcasted_iota(jnp.int32, (8, 1), 0)
        tail = jnp.where(rows8 < r, v[:8, :], u16[8:16, :])
        s_ref[pl.ds(FEAT_LEN, 8), :] = tail

        cp = pltpu.make_async_copy(s_ref, out_ref.at[pl.ds(base, SPAN)], semB)
        cp.start()
        cp.wait()


def kernel(embeddings, feature_0, feature_1, feature_2, feature_3, multimodal_locs):
    return pl.pallas_call(
        _body,
        in_specs=[
            pl.BlockSpec(memory_space=pltpu.SMEM),
            pl.BlockSpec(memory_space=pltpu.MemorySpace.HBM),
            pl.BlockSpec(memory_space=pltpu.MemorySpace.HBM),
            pl.BlockSpec(memory_space=pltpu.MemorySpace.HBM),
            pl.BlockSpec(memory_space=pltpu.MemorySpace.HBM),
            pl.BlockSpec(memory_space=pltpu.MemorySpace.HBM),
        ],
        out_specs=pl.BlockSpec(memory_space=pltpu.MemorySpace.HBM),
        out_shape=jax.ShapeDtypeStruct((TOKENS, HIDDEN), jnp.float32),
        scratch_shapes=[
            pltpu.VMEM((SPAN + 8, HIDDEN), jnp.float32),
            pltpu.VMEM((SPAN, HIDDEN), jnp.float32),
            pltpu.SemaphoreType.DMA((NCHUNK,)),
            pltpu.SemaphoreType.DMA,
        ],
    )(multimodal_locs, embeddings, feature_0, feature_1, feature_2, feature_3)


# speedup vs baseline: 18.3560x; 18.3560x over previous
import jax
import jax.numpy as jnp
from jax import lax
from jax.experimental import pallas as pl
from jax.experimental.pallas import tpu as pltpu

TOKENS = 32768
HIDDEN = 2048
FEAT_LEN = 1024
NUM_FEATS = 4
COPY_BLOCK = 512
NTILE = FEAT_LEN // 8 + 1


def _copy_body(emb_ref, out_ref):
    out_ref[...] = emb_ref[...]


def _feat_body(i, locs_s, fb_ref, cur_hbm, out_ref, prev, e0, e1):
    t = pl.program_id(0)
    loc = locs_s[i]
    base = pl.multiple_of((loc // 8) * 8, 8)
    r = loc - base

    @pl.when(t == 0)
    def _():
        pltpu.sync_copy(cur_hbm.at[pl.ds(base, 8)], e0)
        pltpu.sync_copy(cur_hbm.at[pl.ds(base + FEAT_LEN, 8)], e1)

    fb = fb_ref[...]
    fa = jnp.where(t == 0, fb, prev[...])
    u16 = jnp.concatenate([fa, fb], axis=0)
    v = pltpu.roll(u16, 8 + r, 0)[:8, :]
    g = base + 8 * t + lax.broadcasted_iota(jnp.int32, (8, 1), 0)
    infeat = (g >= loc) & (g < loc + FEAT_LEN)
    cur_tile = jnp.where(t == 0, e0[...], e1[...])
    out_ref[...] = jnp.where(infeat, v, cur_tile)
    prev[...] = fb


def kernel(embeddings, feature_0, feature_1, feature_2, feature_3, multimodal_locs):
    locs = multimodal_locs.astype(jnp.int32)

    out = pl.pallas_call(
        _copy_body,
        grid=(TOKENS // COPY_BLOCK,),
        in_specs=[pl.BlockSpec((COPY_BLOCK, HIDDEN), lambda c: (c, 0))],
        out_specs=pl.BlockSpec((COPY_BLOCK, HIDDEN), lambda c: (c, 0)),
        out_shape=jax.ShapeDtypeStruct((TOKENS, HIDDEN), jnp.float32),
    )(embeddings)

    feats = [feature_0, feature_1, feature_2, feature_3]
    for i in range(NUM_FEATS):
        grid_spec = pltpu.PrefetchScalarGridSpec(
            num_scalar_prefetch=1,
            grid=(NTILE,),
            in_specs=[
                pl.BlockSpec(
                    (8, HIDDEN),
                    lambda t, locs_ref: (jnp.minimum(t, FEAT_LEN // 8 - 1), 0),
                ),
                pl.BlockSpec(memory_space=pltpu.MemorySpace.HBM),
            ],
            out_specs=pl.BlockSpec(
                (8, HIDDEN),
                lambda t, locs_ref, i=i: (locs_ref[i] // 8 + t, 0),
            ),
            scratch_shapes=[
                pltpu.VMEM((8, HIDDEN), jnp.float32),
                pltpu.VMEM((8, HIDDEN), jnp.float32),
                pltpu.VMEM((8, HIDDEN), jnp.float32),
            ],
        )
        out = pl.pallas_call(
            lambda *a, i=i: _feat_body(i, *a),
            grid_spec=grid_spec,
            out_shape=jax.ShapeDtypeStruct((TOKENS, HIDDEN), jnp.float32),
            input_output_aliases={2: 0},
        )(locs, feats[i], out)
    return out


# speedup vs baseline: 34.8009x; 1.8959x over previous
import jax
import jax.numpy as jnp
from jax import lax
from jax.experimental import pallas as pl
from jax.experimental.pallas import tpu as pltpu

TOKENS = 32768
HIDDEN = 2048
FEAT_LEN = 1024
NUM_FEATS = 4
COPY_BLOCK = 1024
FB = 128
NTILE = FEAT_LEN // FB + 1


def _copy_body(emb_ref, out_ref):
    out_ref[...] = emb_ref[...]


def _feat_body(i, locs_s, fb_ref, cur_hbm, out_ref, prev, e0, e1):
    t = pl.program_id(0)
    loc = locs_s[i]
    base = pl.multiple_of((loc // FB) * FB, FB)
    r = loc - base

    @pl.when(t == 0)
    def _():
        pltpu.sync_copy(cur_hbm.at[pl.ds(base, FB)], e0)
        pltpu.sync_copy(cur_hbm.at[pl.ds(base + FEAT_LEN, FB)], e1)

    fb = fb_ref[...]
    fa = jnp.where(t == 0, fb, prev[...])
    u = jnp.concatenate([fa, fb], axis=0)
    v = pltpu.roll(u, FB + r, 0)[:FB, :]
    g = base + FB * t + lax.broadcasted_iota(jnp.int32, (FB, 1), 0)
    infeat = (g >= loc) & (g < loc + FEAT_LEN)
    cur_tile = jnp.where(t == 0, e0[...], e1[...])
    out_ref[...] = jnp.where(infeat, v, cur_tile)
    prev[...] = fb


def kernel(embeddings, feature_0, feature_1, feature_2, feature_3, multimodal_locs):
    locs = multimodal_locs.astype(jnp.int32)

    out = pl.pallas_call(
        _copy_body,
        grid=(TOKENS // COPY_BLOCK,),
        in_specs=[pl.BlockSpec((COPY_BLOCK, HIDDEN), lambda c: (c, 0))],
        out_specs=pl.BlockSpec((COPY_BLOCK, HIDDEN), lambda c: (c, 0)),
        out_shape=jax.ShapeDtypeStruct((TOKENS, HIDDEN), jnp.float32),
    )(embeddings)

    feats = [feature_0, feature_1, feature_2, feature_3]
    for i in range(NUM_FEATS):
        grid_spec = pltpu.PrefetchScalarGridSpec(
            num_scalar_prefetch=1,
            grid=(NTILE,),
            in_specs=[
                pl.BlockSpec(
                    (FB, HIDDEN),
                    lambda t, locs_ref: (jnp.minimum(t, FEAT_LEN // FB - 1), 0),
                ),
                pl.BlockSpec(memory_space=pltpu.MemorySpace.HBM),
            ],
            out_specs=pl.BlockSpec(
                (FB, HIDDEN),
                lambda t, locs_ref, i=i: (locs_ref[i] // FB + t, 0),
            ),
            scratch_shapes=[
                pltpu.VMEM((FB, HIDDEN), jnp.float32),
                pltpu.VMEM((FB, HIDDEN), jnp.float32),
                pltpu.VMEM((FB, HIDDEN), jnp.float32),
            ],
        )
        out = pl.pallas_call(
            lambda *a, i=i: _feat_body(i, *a),
            grid_spec=grid_spec,
            out_shape=jax.ShapeDtypeStruct((TOKENS, HIDDEN), jnp.float32),
            input_output_aliases={2: 0},
        )(locs, feats[i], out)
    return out
